# Initial kernel scaffold; baseline (speedup 1.0000x reference)
import jax
import jax.numpy as jnp
from jax.experimental import pallas as pl


def kernel(hidden_states, Wg, W1, W3, W2):
    raise NotImplementedError("write your pallas kernel here")


# speedup vs baseline: 2.1157x; 2.1157x over previous
import functools

import jax
import jax.numpy as jnp
from jax import lax
from jax.experimental import pallas as pl
from jax.experimental.pallas import tpu as pltpu

E = 8
TOP_K = 2
H = 1024
I = 4096

B = 512
MAXB = I // B + E
NI = 4
TI = I // NI
PADN = 2 * 2048 + B


def _routing_body(x_ref, wg_ref, comb_ref):
    x = x_ref[...]
    logits = lax.dot_general(x, wg_ref[...], (((1,), (1,)), ((), ())),
                             preferred_element_type=jnp.float32)
    logits = 30.0 * jnp.tanh(logits / 30.0)
    m = jnp.max(logits, axis=1, keepdims=True)
    e = jnp.exp(logits - m)
    probs = e / jnp.sum(e, axis=1, keepdims=True)
    iot = lax.broadcasted_iota(jnp.int32, probs.shape, 1)
    m1 = jnp.max(probs, axis=1, keepdims=True)
    i1 = jnp.min(jnp.where(probs == m1, iot, E), axis=1, keepdims=True)
    masked = jnp.where(iot == i1, -jnp.inf, probs)
    m2 = jnp.max(masked, axis=1, keepdims=True)
    i2 = jnp.min(jnp.where(masked == m2, iot, E), axis=1, keepdims=True)
    s = m1 + m2
    comb_ref[...] = jnp.where(iot == i1, m1 / s,
                              jnp.where(iot == i2, m2 / s, 0.0))


def _moe_body(be_ref, blen_ref, rows_ref, w_ref, x_ref, w1_ref, w3_ref,
              w2_ref, out_ref):
    i = pl.program_id(0)
    j = pl.program_id(1)

    @pl.when(jnp.logical_and(i == 0, j == 0))
    def _():
        out_ref[...] = jnp.zeros_like(out_ref)

    @pl.when(blen_ref[j] > 0)
    def _():
        rows = rows_ref[0, 0, :]
        wvec = w_ref[0, 0, :]
        tio = lax.broadcasted_iota(jnp.int32, (x_ref.shape[0], B), 0)
        onehot = tio == rows[None, :]
        pt = jnp.where(onehot, 1.0, 0.0)
        xb = lax.dot_general(pt, x_ref[...], (((0,), (0,)), ((), ())),
                             preferred_element_type=jnp.float32)
        h1 = lax.dot_general(xb, w1_ref[0], (((1,), (1,)), ((), ())),
                             preferred_element_type=jnp.float32)
        u = lax.dot_general(xb, w3_ref[0], (((1,), (1,)), ((), ())),
                            preferred_element_type=jnp.float32)
        g = h1 * 0.5 * (1.0 + lax.erf(h1 * 0.7071067811865476))
        act = g * u
        contrib = lax.dot_general(act, w2_ref[0], (((1,), (1,)), ((), ())),
                                  preferred_element_type=jnp.float32)
        ptw = jnp.where(onehot, wvec[None, :], 0.0)
        out_ref[...] += lax.dot_general(ptw, contrib,
                                        (((1,), (0,)), ((), ())),
                                        preferred_element_type=jnp.float32)


def kernel(hidden_states, Wg, W1, W3, W2):
    orig_shape = hidden_states.shape
    x = hidden_states.reshape(-1, H)
    T = x.shape[0]

    comb = pl.pallas_call(
        _routing_body,
        out_shape=jax.ShapeDtypeStruct((T, E), jnp.float32),
    )(x, Wg)

    i32 = jnp.int32
    mask = comb > 0.0
    counts = jnp.sum(mask.astype(i32), axis=0)
    starts = jnp.concatenate(
        [jnp.zeros((1,), i32), jnp.cumsum(counts)[:-1].astype(i32)])
    within = jnp.cumsum(mask.astype(i32), axis=0) - 1
    pos = starts[None, :] + within
    posf = jnp.where(mask, pos, PADN).reshape(-1)
    tokf = jnp.broadcast_to(jnp.arange(T, dtype=i32)[:, None], (T, E)).reshape(-1)
    srows = jnp.zeros((PADN,), i32).at[posf].set(tokf, mode="drop")
    sw = jnp.zeros((PADN,), jnp.float32).at[posf].set(
        comb.reshape(-1), mode="drop")

    nblk = (counts + B - 1) // B
    cumblk = jnp.cumsum(nblk).astype(i32)
    jidx = jnp.arange(MAXB, dtype=i32)
    bexp = jnp.searchsorted(cumblk, jidx, side="right").astype(i32)
    bec = jnp.minimum(bexp, E - 1)
    prev = jnp.where(bec > 0, cumblk[jnp.maximum(bec - 1, 0)], 0)
    kk = jidx - prev
    bstart = starts[bec] + kk * B
    blen = jnp.clip(counts[bec] - kk * B, 0, B).astype(i32)

    ridx = bstart[:, None] + jnp.arange(B, dtype=i32)[None, :]
    valid = jnp.arange(B, dtype=i32)[None, :] < blen[:, None]
    ridxc = jnp.clip(ridx, 0, PADN - 1)
    rows2d = jnp.where(valid, srows[ridxc], 0).reshape(MAXB, 1, B)
    w2d = jnp.where(valid, sw[ridxc], 0.0).reshape(MAXB, 1, B)

    grid_spec = pltpu.PrefetchScalarGridSpec(
        num_scalar_prefetch=2,
        grid=(NI, MAXB),
        in_specs=[
            pl.BlockSpec((1, 1, B), lambda i, j, be, bl: (j, 0, 0)),
            pl.BlockSpec((1, 1, B), lambda i, j, be, bl: (j, 0, 0)),
            pl.BlockSpec((T, H), lambda i, j, be, bl: (0, 0)),
            pl.BlockSpec((1, TI, H), lambda i, j, be, bl: (be[j], i, 0)),
            pl.BlockSpec((1, TI, H), lambda i, j, be, bl: (be[j], i, 0)),
            pl.BlockSpec((1, H, TI), lambda i, j, be, bl: (be[j], 0, i)),
        ],
        out_specs=pl.BlockSpec((T, H), lambda i, j, be, bl: (0, 0)),
    )
    out = pl.pallas_call(
        _moe_body,
        grid_spec=grid_spec,
        out_shape=jax.ShapeDtypeStruct((T, H), jnp.float32),
    )(bec, blen, rows2d, w2d, x, W1, W3, W2)

    return out.reshape(orig_shape)


# speedup vs baseline: 2.4226x; 1.1451x over previous
import functools

import jax
import jax.numpy as jnp
from jax import lax
from jax.experimental import pallas as pl
from jax.experimental.pallas import tpu as pltpu

E = 8
TOP_K = 2
H = 1024
I = 4096

B = 512
MAXB = I // B + E
NI = 4
TI = I // NI
PADN = 2 * 2048 + B


def _routing_body(x_ref, wg_ref, comb_ref):
    x = x_ref[...]
    logits = lax.dot_general(x, wg_ref[...], (((1,), (1,)), ((), ())),
                             preferred_element_type=jnp.float32)
    logits = 30.0 * jnp.tanh(logits / 30.0)
    m = jnp.max(logits, axis=1, keepdims=True)
    e = jnp.exp(logits - m)
    probs = e / jnp.sum(e, axis=1, keepdims=True)
    iot = lax.broadcasted_iota(jnp.int32, probs.shape, 1)
    m1 = jnp.max(probs, axis=1, keepdims=True)
    i1 = jnp.min(jnp.where(probs == m1, iot, E), axis=1, keepdims=True)
    masked = jnp.where(iot == i1, -jnp.inf, probs)
    m2 = jnp.max(masked, axis=1, keepdims=True)
    i2 = jnp.min(jnp.where(masked == m2, iot, E), axis=1, keepdims=True)
    s = m1 + m2
    comb_ref[...] = jnp.where(iot == i1, m1 / s,
                              jnp.where(iot == i2, m2 / s, 0.0))


def _moe_body(be_ref, blen_ref, rows_ref, w_ref, x_ref, w1_ref, w3_ref,
              w2_ref, out_ref, xb_ref, acc_ref):
    j = pl.program_id(0)
    i = pl.program_id(1)

    @pl.when(jnp.logical_and(i == 0, j == 0))
    def _():
        out_ref[...] = jnp.zeros_like(out_ref)

    @pl.when(blen_ref[j] > 0)
    def _():
        rows = rows_ref[0, 0, :]

        @pl.when(i == 0)
        def _():
            tio = lax.broadcasted_iota(jnp.int32, (x_ref.shape[0], B), 0)
            pt = jnp.where(tio == rows[None, :], 1.0, 0.0)
            xb_ref[...] = lax.dot_general(pt, x_ref[...],
                                          (((0,), (0,)), ((), ())),
                                          preferred_element_type=jnp.float32)

        xb = xb_ref[...]
        h1 = lax.dot_general(xb, w1_ref[0], (((1,), (1,)), ((), ())),
                             preferred_element_type=jnp.float32)
        u = lax.dot_general(xb, w3_ref[0], (((1,), (1,)), ((), ())),
                            preferred_element_type=jnp.float32)
        g = h1 * 0.5 * (1.0 + lax.erf(h1 * 0.7071067811865476))
        act = g * u
        contrib = lax.dot_general(act, w2_ref[0], (((1,), (1,)), ((), ())),
                                  preferred_element_type=jnp.float32)

        @pl.when(i == 0)
        def _():
            acc_ref[...] = contrib

        @pl.when(i > 0)
        def _():
            acc_ref[...] += contrib

        @pl.when(i == NI - 1)
        def _():
            wvec = w_ref[0, 0, :]
            tio = lax.broadcasted_iota(jnp.int32, (x_ref.shape[0], B), 0)
            ptw = jnp.where(tio == rows[None, :], wvec[None, :], 0.0)
            out_ref[...] += lax.dot_general(ptw, acc_ref[...],
                                            (((1,), (0,)), ((), ())),
                                            preferred_element_type=jnp.float32)


def kernel(hidden_states, Wg, W1, W3, W2):
    orig_shape = hidden_states.shape
    x = hidden_states.reshape(-1, H)
    T = x.shape[0]

    comb = pl.pallas_call(
        _routing_body,
        out_shape=jax.ShapeDtypeStruct((T, E), jnp.float32),
    )(x, Wg)

    i32 = jnp.int32
    mask = comb > 0.0
    counts = jnp.sum(mask.astype(i32), axis=0)
    starts = jnp.concatenate(
        [jnp.zeros((1,), i32), jnp.cumsum(counts)[:-1].astype(i32)])
    within = jnp.cumsum(mask.astype(i32), axis=0) - 1
    pos = starts[None, :] + within
    posf = jnp.where(mask, pos, PADN).reshape(-1)
    tokf = jnp.broadcast_to(jnp.arange(T, dtype=i32)[:, None], (T, E)).reshape(-1)
    srows = jnp.zeros((PADN,), i32).at[posf].set(tokf, mode="drop")
    sw = jnp.zeros((PADN,), jnp.float32).at[posf].set(
        comb.reshape(-1), mode="drop")

    nblk = (counts + B - 1) // B
    cumblk = jnp.cumsum(nblk).astype(i32)
    jidx = jnp.arange(MAXB, dtype=i32)
    bexp = jnp.searchsorted(cumblk, jidx, side="right").astype(i32)
    bec = jnp.minimum(bexp, E - 1)
    prev = jnp.where(bec > 0, cumblk[jnp.maximum(bec - 1, 0)], 0)
    kk = jidx - prev
    bstart = starts[bec] + kk * B
    blen = jnp.clip(counts[bec] - kk * B, 0, B).astype(i32)

    ridx = bstart[:, None] + jnp.arange(B, dtype=i32)[None, :]
    valid = jnp.arange(B, dtype=i32)[None, :] < blen[:, None]
    ridxc = jnp.clip(ridx, 0, PADN - 1)
    rows2d = jnp.where(valid, srows[ridxc], 0).reshape(MAXB, 1, B)
    w2d = jnp.where(valid, sw[ridxc], 0.0).reshape(MAXB, 1, B)

    grid_spec = pltpu.PrefetchScalarGridSpec(
        num_scalar_prefetch=2,
        grid=(MAXB, NI),
        in_specs=[
            pl.BlockSpec((1, 1, B), lambda j, i, be, bl: (j, 0, 0)),
            pl.BlockSpec((1, 1, B), lambda j, i, be, bl: (j, 0, 0)),
            pl.BlockSpec((T, H), lambda j, i, be, bl: (0, 0)),
            pl.BlockSpec((1, TI, H), lambda j, i, be, bl: (be[j], i, 0)),
            pl.BlockSpec((1, TI, H), lambda j, i, be, bl: (be[j], i, 0)),
            pl.BlockSpec((1, H, TI), lambda j, i, be, bl: (be[j], 0, i)),
        ],
        out_specs=pl.BlockSpec((T, H), lambda j, i, be, bl: (0, 0)),
        scratch_shapes=[
            pltpu.VMEM((B, H), jnp.float32),
            pltpu.VMEM((B, H), jnp.float32),
        ],
    )
    out = pl.pallas_call(
        _moe_body,
        grid_spec=grid_spec,
        out_shape=jax.ShapeDtypeStruct((T, H), jnp.float32),
    )(bec, blen, rows2d, w2d, x, W1, W3, W2)

    return out.reshape(orig_shape)


# speedup vs baseline: 2.4277x; 1.0021x over previous
import functools

import jax
import jax.numpy as jnp
from jax import lax
from jax.experimental import pallas as pl
from jax.experimental.pallas import tpu as pltpu

E = 8
TOP_K = 2
H = 1024
I = 4096

B = 512
MAXB = I // B + E
NI = 4
TI = I // NI
PADN = 2 * 2048 + B


def _routing_body(x_ref, wg_ref, comb_ref):
    x = x_ref[...]
    logits = lax.dot_general(x, wg_ref[...], (((1,), (1,)), ((), ())),
                             preferred_element_type=jnp.float32)
    logits = 30.0 * jnp.tanh(logits / 30.0)
    m = jnp.max(logits, axis=1, keepdims=True)
    e = jnp.exp(logits - m)
    probs = e / jnp.sum(e, axis=1, keepdims=True)
    iot = lax.broadcasted_iota(jnp.int32, probs.shape, 1)
    m1 = jnp.max(probs, axis=1, keepdims=True)
    i1 = jnp.min(jnp.where(probs == m1, iot, E), axis=1, keepdims=True)
    masked = jnp.where(iot == i1, -jnp.inf, probs)
    m2 = jnp.max(masked, axis=1, keepdims=True)
    i2 = jnp.min(jnp.where(masked == m2, iot, E), axis=1, keepdims=True)
    s = m1 + m2
    comb_ref[...] = jnp.where(iot == i1, m1 / s,
                              jnp.where(iot == i2, m2 / s, 0.0))


def _moe_body(be_ref, blen_ref, rows_ref, w_ref, x_ref, w1_ref, w3_ref,
              w2_ref, out_ref, xb_ref, acc_ref):
    j = pl.program_id(0)
    i = pl.program_id(1)

    @pl.when(jnp.logical_and(i == 0, j == 0))
    def _():
        out_ref[...] = jnp.zeros_like(out_ref)

    @pl.when(blen_ref[j] > 0)
    def _():
        rows = rows_ref[0, 0, :]

        @pl.when(i == 0)
        def _():
            tio = lax.broadcasted_iota(jnp.int32, (x_ref.shape[0], B), 0)
            pt = jnp.where(tio == rows[None, :], 1.0, 0.0).astype(jnp.bfloat16)
            xb_ref[...] = lax.dot_general(pt, x_ref[...].astype(jnp.bfloat16),
                                          (((0,), (0,)), ((), ())),
                                          preferred_element_type=jnp.float32)

        xb = xb_ref[...].astype(jnp.bfloat16)
        h1 = lax.dot_general(xb, w1_ref[0].astype(jnp.bfloat16),
                             (((1,), (1,)), ((), ())),
                             preferred_element_type=jnp.float32)
        u = lax.dot_general(xb, w3_ref[0].astype(jnp.bfloat16),
                            (((1,), (1,)), ((), ())),
                            preferred_element_type=jnp.float32)
        g = h1 * 0.5 * (1.0 + lax.erf(h1 * 0.7071067811865476))
        act = (g * u).astype(jnp.bfloat16)
        contrib = lax.dot_general(act, w2_ref[0].astype(jnp.bfloat16),
                                  (((1,), (1,)), ((), ())),
                                  preferred_element_type=jnp.float32)

        @pl.when(i == 0)
        def _():
            acc_ref[...] = contrib

        @pl.when(i > 0)
        def _():
            acc_ref[...] += contrib

        @pl.when(i == NI - 1)
        def _():
            wvec = w_ref[0, 0, :]
            tio = lax.broadcasted_iota(jnp.int32, (x_ref.shape[0], B), 0)
            ptw = jnp.where(tio == rows[None, :], wvec[None, :],
                            0.0).astype(jnp.bfloat16)
            out_ref[...] += lax.dot_general(ptw,
                                            acc_ref[...].astype(jnp.bfloat16),
                                            (((1,), (0,)), ((), ())),
                                            preferred_element_type=jnp.float32)


def kernel(hidden_states, Wg, W1, W3, W2):
    orig_shape = hidden_states.shape
    x = hidden_states.reshape(-1, H)
    T = x.shape[0]

    comb = pl.pallas_call(
        _routing_body,
        out_shape=jax.ShapeDtypeStruct((T, E), jnp.float32),
    )(x, Wg)

    i32 = jnp.int32
    mask = comb > 0.0
    counts = jnp.sum(mask.astype(i32), axis=0)
    starts = jnp.concatenate(
        [jnp.zeros((1,), i32), jnp.cumsum(counts)[:-1].astype(i32)])
    within = jnp.cumsum(mask.astype(i32), axis=0) - 1
    pos = starts[None, :] + within
    posf = jnp.where(mask, pos, PADN).reshape(-1)
    tokf = jnp.broadcast_to(jnp.arange(T, dtype=i32)[:, None], (T, E)).reshape(-1)
    srows = jnp.zeros((PADN,), i32).at[posf].set(tokf, mode="drop")
    sw = jnp.zeros((PADN,), jnp.float32).at[posf].set(
        comb.reshape(-1), mode="drop")

    nblk = (counts + B - 1) // B
    cumblk = jnp.cumsum(nblk).astype(i32)
    jidx = jnp.arange(MAXB, dtype=i32)
    bexp = jnp.searchsorted(cumblk, jidx, side="right").astype(i32)
    bec = jnp.minimum(bexp, E - 1)
    prev = jnp.where(bec > 0, cumblk[jnp.maximum(bec - 1, 0)], 0)
    kk = jidx - prev
    bstart = starts[bec] + kk * B
    blen = jnp.clip(counts[bec] - kk * B, 0, B).astype(i32)

    ridx = bstart[:, None] + jnp.arange(B, dtype=i32)[None, :]
    valid = jnp.arange(B, dtype=i32)[None, :] < blen[:, None]
    ridxc = jnp.clip(ridx, 0, PADN - 1)
    rows2d = jnp.where(valid, srows[ridxc], 0).reshape(MAXB, 1, B)
    w2d = jnp.where(valid, sw[ridxc], 0.0).reshape(MAXB, 1, B)

    grid_spec = pltpu.PrefetchScalarGridSpec(
        num_scalar_prefetch=2,
        grid=(MAXB, NI),
        in_specs=[
            pl.BlockSpec((1, 1, B), lambda j, i, be, bl: (j, 0, 0)),
            pl.BlockSpec((1, 1, B), lambda j, i, be, bl: (j, 0, 0)),
            pl.BlockSpec((T, H), lambda j, i, be, bl: (0, 0)),
            pl.BlockSpec((1, TI, H), lambda j, i, be, bl: (be[j], i, 0)),
            pl.BlockSpec((1, TI, H), lambda j, i, be, bl: (be[j], i, 0)),
            pl.BlockSpec((1, H, TI), lambda j, i, be, bl: (be[j], 0, i)),
        ],
        out_specs=pl.BlockSpec((T, H), lambda j, i, be, bl: (0, 0)),
        scratch_shapes=[
            pltpu.VMEM((B, H), jnp.float32),
            pltpu.VMEM((B, H), jnp.float32),
        ],
    )
    out = pl.pallas_call(
        _moe_body,
        grid_spec=grid_spec,
        out_shape=jax.ShapeDtypeStruct((T, H), jnp.float32),
    )(bec, blen, rows2d, w2d, x, W1, W3, W2)

    return out.reshape(orig_shape)
